# baseline (speedup 1.0000x reference)
import functools

import jax
import jax.numpy as jnp
from jax import lax
from jax.experimental import pallas as pl
from jax.experimental.pallas import tpu as pltpu
from jax.experimental.pallas import tpu_sc as plsc

VOCAB = 1000000
DIM = 64
B = 4096
L = 50
W16 = 2 * DIM

_info = plsc.get_sparse_core_info()
NC, NS = _info.num_cores, _info.num_subcores
NW = NC * NS
TOTAL = B * L
CHUNK = 128
NCHUNK = TOTAL // (NW * CHUNK)

_mesh = plsc.VectorSubcoreMesh(core_axis_name="c", subcore_axis_name="s")


@functools.partial(
    pl.kernel,
    mesh=_mesh,
    out_type=jax.ShapeDtypeStruct((TOTAL, W16), jnp.float32),
    scratch_types=[
        pltpu.VMEM((NCHUNK, CHUNK), jnp.int32),
        pltpu.VMEM((2, CHUNK, W16), jnp.float32),
        pltpu.SemaphoreType.DMA((2,)),
        pltpu.SemaphoreType.DMA((2,)),
    ],
    compiler_params=pltpu.CompilerParams(use_tc_tiling_on_sc=True),
)
def _gather(table_hbm, idx_hbm, out_hbm, idx_v, rows_v, sem_g, sem_w):
    wid = lax.axis_index("s") * NC + lax.axis_index("c")
    pltpu.sync_copy(idx_hbm.at[wid], idx_v)

    def gather_chunk(g, bb):
        return pltpu.make_async_copy(
            table_hbm.at[idx_v.at[g]], rows_v.at[bb], sem_g.at[bb]
        )

    def write_chunk(g, bb):
        return pltpu.make_async_copy(
            rows_v.at[bb],
            out_hbm.at[pl.ds(wid * NCHUNK * CHUNK + g * CHUNK, CHUNK)],
            sem_w.at[bb],
        )

    gather_chunk(0, 0).start()

    def step(g, carry):
        bb = lax.rem(g, 2)
        pb = 1 - bb

        @pl.when(g >= 2)
        def _():
            write_chunk(g - 2, bb).wait()

        gather_chunk(g, bb).start()
        gather_chunk(g - 1, pb).wait()
        write_chunk(g - 1, pb).start()
        return carry

    lax.fori_loop(1, NCHUNK, step, 0)

    last = NCHUNK - 1
    lb = last % 2
    write_chunk(last - 1, 1 - lb).wait()
    gather_chunk(last, lb).wait()
    wlast = write_chunk(last, lb)
    wlast.start()
    wlast.wait()


def kernel(input_, weight):
    pairs = weight.reshape(VOCAB // 2, W16)
    flat_idx = input_.reshape(TOTAL).astype(jnp.int32)
    idx3 = (flat_idx // 2).reshape(NW, NCHUNK, CHUNK)
    out2 = _gather(pairs, idx3).reshape(TOTAL, 2, DIM)
    out = jnp.take_along_axis(
        out2, (flat_idx % 2)[:, None, None], axis=1
    )
    return out.reshape(B, L, DIM)


# speedup vs baseline: 1.4788x; 1.4788x over previous
import functools

import jax
import jax.numpy as jnp
from jax import lax
from jax.experimental import pallas as pl
from jax.experimental.pallas import tpu as pltpu
from jax.experimental.pallas import tpu_sc as plsc

VOCAB = 1000000
DIM = 64
B = 4096
L = 50
WIDE = 2 * DIM

_info = plsc.get_sparse_core_info()
NC, NS = _info.num_cores, _info.num_subcores
NW = NC * NS
TOTAL = B * L
CHUNK = 128
NCHUNK = TOTAL // (NW * CHUNK)

_mesh = plsc.VectorSubcoreMesh(core_axis_name="c", subcore_axis_name="s")


@functools.partial(
    pl.kernel,
    mesh=_mesh,
    out_type=jax.ShapeDtypeStruct((TOTAL, WIDE), jnp.float32),
    scratch_types=[
        pltpu.VMEM((NCHUNK, CHUNK), jnp.int32),
        pltpu.VMEM((2, CHUNK, WIDE), jnp.float32),
        pltpu.SemaphoreType.DMA((2,)),
        pltpu.SemaphoreType.DMA((2,)),
    ],
    compiler_params=pltpu.CompilerParams(use_tc_tiling_on_sc=True),
)
def _gather(table_hbm, idx_hbm, out_hbm, idx_v, rows_v, sem_g, sem_w):
    wid = lax.axis_index("s") * NC + lax.axis_index("c")
    pltpu.sync_copy(idx_hbm.at[wid], idx_v)

    def gather_chunk(g, bb):
        return pltpu.make_async_copy(
            table_hbm.at[idx_v.at[g]], rows_v.at[bb], sem_g.at[bb]
        )

    def write_chunk(g, bb):
        return pltpu.make_async_copy(
            rows_v.at[bb],
            out_hbm.at[pl.ds(wid * NCHUNK * CHUNK + g * CHUNK, CHUNK)],
            sem_w.at[bb],
        )

    gather_chunk(0, 0).start()

    def step(g, carry):
        bb = lax.rem(g, 2)
        pb = 1 - bb

        @pl.when(g >= 2)
        def _():
            write_chunk(g - 2, bb).wait()

        gather_chunk(g, bb).start()
        gather_chunk(g - 1, pb).wait()
        write_chunk(g - 1, pb).start()
        return carry

    lax.fori_loop(1, NCHUNK, step, 0)

    last = NCHUNK - 1
    lb = last % 2
    write_chunk(last - 1, 1 - lb).wait()
    gather_chunk(last, lb).wait()
    wlast = write_chunk(last, lb)
    wlast.start()
    wlast.wait()


def kernel(input_, weight):
    wide = jnp.concatenate([weight, weight], axis=1)
    idx3 = input_.reshape(NW, NCHUNK, CHUNK).astype(jnp.int32)
    out = _gather(wide, idx3)
    return out[:, :DIM].reshape(B, L, DIM)


# speedup vs baseline: 2.0406x; 1.3799x over previous
import functools

import jax
import jax.numpy as jnp
from jax import lax
from jax.experimental import pallas as pl
from jax.experimental.pallas import tpu as pltpu
from jax.experimental.pallas import tpu_sc as plsc

VOCAB = 1000000
DIM = 64
B = 4096
L = 50
WIDE = 2 * DIM

_info = plsc.get_sparse_core_info()
NC, NS = _info.num_cores, _info.num_subcores
NW = NC * NS
TOTAL = B * L
CHUNK = 128
NCHUNK = TOTAL // (NW * CHUNK)

_mesh = plsc.VectorSubcoreMesh(core_axis_name="c", subcore_axis_name="s")


@functools.partial(
    pl.kernel,
    mesh=_mesh,
    out_type=jax.ShapeDtypeStruct((B, L, WIDE), jnp.float32),
    scratch_types=[
        pltpu.VMEM((NCHUNK, CHUNK), jnp.int32),
        pltpu.VMEM((2, CHUNK, WIDE), jnp.float32),
        pltpu.SemaphoreType.DMA((2,)),
        pltpu.SemaphoreType.DMA((2,)),
    ],
    compiler_params=pltpu.CompilerParams(use_tc_tiling_on_sc=True),
)
def _gather(table_hbm, idx_hbm, out_hbm, idx_v, rows_v, sem_g, sem_w):
    wid = lax.axis_index("s") * NC + lax.axis_index("c")
    pltpu.sync_copy(idx_hbm.at[wid], idx_v)

    def gather_chunk(g, bb):
        return pltpu.make_async_copy(
            table_hbm.at[idx_v.at[g]], rows_v.at[bb], sem_g.at[bb]
        )

    def write_chunk(g, bb):
        return pltpu.make_async_copy(
            rows_v.at[bb],
            out_hbm.at[pl.ds(wid * CHUNK, CHUNK), g],
            sem_w.at[bb],
        )

    gather_chunk(0, 0).start()

    def step(g, carry):
        bb = lax.rem(g, 2)
        pb = 1 - bb

        @pl.when(g >= 2)
        def _():
            write_chunk(g - 2, bb).wait()

        gather_chunk(g, bb).start()
        gather_chunk(g - 1, pb).wait()
        write_chunk(g - 1, pb).start()
        return carry

    lax.fori_loop(1, NCHUNK, step, 0)

    last = NCHUNK - 1
    lb = last % 2
    write_chunk(last - 1, 1 - lb).wait()
    gather_chunk(last, lb).wait()
    wlast = write_chunk(last, lb)
    wlast.start()
    wlast.wait()


def kernel(input_, weight):
    wide = jnp.pad(weight, ((0, 0), (0, DIM)))
    idx3 = input_.reshape(NW, CHUNK, L).transpose(0, 2, 1).astype(jnp.int32)
    out = _gather(wide, idx3)
    return out[:, :, :DIM]
